# baseline (speedup 1.0000x reference)
import functools

import jax
import jax.numpy as jnp
from jax import lax
from jax.experimental import pallas as pl
from jax.experimental.pallas import tpu as pltpu
from jax.experimental.pallas import tpu_sc as plsc

VOCAB = 100000
D = 768
B = 4
S = 2048
NC = 2
NS = 16
NW = NC * NS
RPW = S // NW
CHUNK = 32
NCHUNK = B * RPW // CHUNK
LANES = 16
VECS_PER_ROW = D // LANES


def _body(ids_hbm, tok_hbm, pos_hbm, out_hbm,
          idx0, idx1, pos_v, tok0, tok1, gs0, gs1, ws0, ws1):
    wid = lax.axis_index("s") * NC + lax.axis_index("c")
    base = wid * RPW

    pltpu.sync_copy(pos_hbm.at[pl.ds(base, RPW)], pos_v)

    idxs = (idx0, idx1)
    bufs = (tok0, tok1)
    gsems = (gs0, gs1)
    wsems = (ws0, ws1)

    def row0_of(c):
        b, half = c // 2, c % 2
        return b * S + base + half * CHUNK

    gh = [None, None]
    wh = [None, None]

    def start_gather(c):
        k = c % 2
        pltpu.sync_copy(ids_hbm.at[pl.ds(row0_of(c), CHUNK)], idxs[k])
        gh[k] = pltpu.make_async_copy(tok_hbm.at[idxs[k]], bufs[k], gsems[k])
        gh[k].start()

    start_gather(0)
    for c in range(NCHUNK):
        cur = c % 2
        if c + 1 < NCHUNK:
            nxt = (c + 1) % 2
            if wh[nxt] is not None:
                wh[nxt].wait()
                wh[nxt] = None
            start_gather(c + 1)
        gh[cur].wait()

        buf = bufs[cur]
        pos_off = (c % 2) * CHUNK

        def add_row(r, carry, buf=buf, pos_off=pos_off):
            tv = buf.at[r]
            pv = pos_v.at[pos_off + r]
            for j in range(VECS_PER_ROW):
                sl = pl.ds(j * LANES, LANES)
                tv[sl] = tv[sl] + pv[sl]
            return carry

        lax.fori_loop(0, CHUNK, add_row, 0)

        wh[cur] = pltpu.make_async_copy(
            buf, out_hbm.at[pl.ds(row0_of(c), CHUNK)], wsems[cur])
        wh[cur].start()

    for k in range(2):
        if wh[k] is not None:
            wh[k].wait()


@functools.partial(jax.jit, static_argnames=())
def _embed(ids_flat, token_table, position_table):
    mesh = plsc.VectorSubcoreMesh(core_axis_name="c", subcore_axis_name="s")
    run = pl.kernel(
        _body,
        out_type=jax.ShapeDtypeStruct((B * S, D), jnp.float32),
        mesh=mesh,
        scratch_types=[
            pltpu.VMEM((CHUNK,), jnp.int32),
            pltpu.VMEM((CHUNK,), jnp.int32),
            pltpu.VMEM((RPW, D), jnp.float32),
            pltpu.VMEM((CHUNK, D), jnp.float32),
            pltpu.VMEM((CHUNK, D), jnp.float32),
            pltpu.SemaphoreType.DMA,
            pltpu.SemaphoreType.DMA,
            pltpu.SemaphoreType.DMA,
            pltpu.SemaphoreType.DMA,
        ],
    )
    return run(ids_flat, token_table, position_table)


def kernel(input_ids, token_table, position_table):
    ids_flat = input_ids.reshape(-1).astype(jnp.int32)
    out = _embed(ids_flat, token_table, position_table)
    return out.reshape(B, S, D)


# speedup vs baseline: 1.1325x; 1.1325x over previous
import functools

import jax
import jax.numpy as jnp
from jax import lax
from jax.experimental import pallas as pl
from jax.experimental.pallas import tpu as pltpu
from jax.experimental.pallas import tpu_sc as plsc

VOCAB = 100000
D = 768
B = 4
S = 2048
NC = 2
NS = 16
NW = NC * NS
RPW = S // NW
CHUNK = 32
NCHUNK = B * RPW // CHUNK
NBUF = 3
LANES = 16
VECS_PER_ROW = D // LANES


def _body(ids_hbm, tok_hbm, pos_hbm, out_hbm,
          idx_all, pos_v, tok0, tok1, tok2, sem_ids, sem_pos,
          gs0, gs1, gs2, ws0, ws1, ws2):
    wid = lax.axis_index("s") * NC + lax.axis_index("c")
    base = wid * RPW

    bufs = (tok0, tok1, tok2)
    gsems = (gs0, gs1, gs2)
    wsems = (ws0, ws1, ws2)

    ph = pltpu.make_async_copy(pos_hbm.at[pl.ds(base, RPW)], pos_v, sem_pos)
    ph.start()
    ih = []
    for b in range(B):
        h = pltpu.make_async_copy(
            ids_hbm.at[pl.ds(b * S + base, RPW)],
            idx_all.at[pl.ds(b * RPW, RPW)], sem_ids)
        h.start()
        ih.append(h)
    for h in ih:
        h.wait()

    def row0_of(c):
        b, half = divmod(c, RPW // CHUNK)
        return b * S + base + half * CHUNK

    gh = [None] * NBUF
    wh = [None] * NBUF

    def start_gather(c):
        k = c % NBUF
        gh[k] = pltpu.make_async_copy(
            tok_hbm.at[idx_all.at[pl.ds(c * CHUNK, CHUNK)]], bufs[k], gsems[k])
        gh[k].start()

    for c in range(NBUF):
        start_gather(c)
    ph.wait()

    for c in range(NCHUNK):
        k = c % NBUF
        gh[c % NBUF].wait()

        buf = bufs[k]
        pos_off = (c % (RPW // CHUNK)) * CHUNK

        def add_row(r, carry, buf=buf, pos_off=pos_off):
            tv = buf.at[r]
            pv = pos_v.at[pos_off + r]
            for j in range(VECS_PER_ROW):
                sl = pl.ds(j * LANES, LANES)
                tv[sl] = tv[sl] + pv[sl]
            return carry

        lax.fori_loop(0, CHUNK, add_row, 0)

        wh[k] = pltpu.make_async_copy(
            buf, out_hbm.at[pl.ds(row0_of(c), CHUNK)], wsems[k])
        wh[k].start()

        if c + NBUF < NCHUNK:
            wh[k].wait()
            wh[k] = None
            start_gather(c + NBUF)

    for k in range(NBUF):
        if wh[k] is not None:
            wh[k].wait()


@functools.partial(jax.jit, static_argnames=())
def _embed(ids_flat, token_table, position_table):
    mesh = plsc.VectorSubcoreMesh(core_axis_name="c", subcore_axis_name="s")
    run = pl.kernel(
        _body,
        out_type=jax.ShapeDtypeStruct((B * S, D), jnp.float32),
        mesh=mesh,
        scratch_types=[
            pltpu.VMEM((B * RPW,), jnp.int32),
            pltpu.VMEM((RPW, D), jnp.float32),
            pltpu.VMEM((CHUNK, D), jnp.float32),
            pltpu.VMEM((CHUNK, D), jnp.float32),
            pltpu.VMEM((CHUNK, D), jnp.float32),
            pltpu.SemaphoreType.DMA,
            pltpu.SemaphoreType.DMA,
            pltpu.SemaphoreType.DMA,
            pltpu.SemaphoreType.DMA,
            pltpu.SemaphoreType.DMA,
            pltpu.SemaphoreType.DMA,
            pltpu.SemaphoreType.DMA,
            pltpu.SemaphoreType.DMA,
        ],
    )
    return run(ids_flat, token_table, position_table)


def kernel(input_ids, token_table, position_table):
    ids_flat = input_ids.reshape(-1).astype(jnp.int32)
    out = _embed(ids_flat, token_table, position_table)
    return out.reshape(B, S, D)
